# baseline (speedup 1.0000x reference)
import functools

import jax
import jax.numpy as jnp
from jax import lax
from jax.experimental import pallas as pl
from jax.experimental.pallas import tpu as pltpu
from jax.experimental.pallas import tpu_sc as plsc

B = 1024
V = 1000
E = 64
NT = 8
SW = 128
NC = 2
NS = 16
L = 16
NW = NC * NS
ROWS = B // NW


def _tc_scores_body(cen_ref, tgt_ref, emb_v_ref, emb_u_ref, av_ref,
                    e_out_ref, av_out_ref, scores_ref):
    rowv = lax.broadcasted_iota(jnp.int32, (V, B), 0)
    oh_c = (cen_ref[...] == rowv).astype(jnp.float32)
    c_b = lax.dot_general(oh_c, emb_v_ref[...], (((0,), (1,)), ((), ())),
                          preferred_element_type=jnp.float32)
    m_b = lax.dot_general(c_b, emb_u_ref[...], (((1,), (0,)), ((), ())),
                          preferred_element_type=jnp.float32)
    tgt_col = jnp.transpose(tgt_ref[...], (1, 0))
    colv = lax.broadcasted_iota(jnp.int32, (B, V), 1)
    scores_ref[...] = jnp.sum(jnp.where(tgt_col == colv, m_b, 0.0),
                              axis=1, keepdims=True)
    e_bt = jnp.exp(m_b)
    av_bt = jnp.transpose(av_ref[...], (1, 0))
    for t in range(NT):
        w = min(SW, V - t * SW)
        e_out_ref[t, :, :w] = e_bt[:, t * SW:t * SW + w]
        av_out_ref[t, :, :w] = av_bt[:, t * SW:t * SW + w]
    e_out_ref[NT - 1, :, V - (NT - 1) * SW:] = jnp.zeros(
        (B, NT * SW - V), jnp.float32)
    av_out_ref[NT - 1, :, V - (NT - 1) * SW:] = jnp.full(
        (B, NT * SW - V), V, jnp.int32)


_tc_scores = pl.pallas_call(
    _tc_scores_body,
    out_shape=(
        jax.ShapeDtypeStruct((NT, B, SW), jnp.float32),
        jax.ShapeDtypeStruct((NT, B, SW), jnp.int32),
        jax.ShapeDtypeStruct((B, 1), jnp.float32),
    ),
)


_sc_mesh = plsc.VectorSubcoreMesh(core_axis_name="c", subcore_axis_name="s")


@functools.partial(
    pl.kernel,
    out_type=jax.ShapeDtypeStruct((B,), jnp.float32),
    mesh=_sc_mesh,
    compiler_params=pltpu.CompilerParams(
        use_tc_tiling_on_sc=False, needs_layout_passes=False),
    scratch_types=[
        pltpu.VMEM((NT, ROWS, SW), jnp.float32),
        pltpu.VMEM((NT, ROWS, SW), jnp.int32),
        pltpu.VMEM((ROWS, L), jnp.float32),
        pltpu.VMEM((ROWS,), jnp.float32),
        pltpu.SemaphoreType.DMA,
        pltpu.SemaphoreType.DMA,
    ],
)
def _sc_gather_sum(e_hbm, av_hbm, out_hbm, e_v, av_v, accs_v, sum_v,
                   sem_e, sem_i):
    wid = lax.axis_index("s") * NC + lax.axis_index("c")
    base = wid * ROWS
    cp_e = pltpu.async_copy(e_hbm.at[:, pl.ds(base, ROWS)], e_v, sem_e)
    cp_i = pltpu.async_copy(av_hbm.at[:, pl.ds(base, ROWS)], av_v, sem_i)
    cp_e.wait()
    cp_i.wait()

    def row_body(r, carry):
        rsplat = jnp.zeros((L,), jnp.int32) + r

        def chunk(k, a, _r=r, _rs=rsplat):
            cols = av_v[k // (SW // L), _r, pl.ds((k % (SW // L)) * L, L)]
            tt = cols >> 7
            cc = cols & 127
            return a + plsc.load_gather(e_v, [tt, _rs, cc])

        acc = lax.fori_loop(0, NT * (SW // L), chunk,
                            jnp.zeros((L,), jnp.float32), unroll=8)
        accs_v[r, :] = acc
        return carry

    lax.fori_loop(0, ROWS, row_body, 0)
    for g in range(ROWS // L):
        rows = lax.iota(jnp.int32, L) + (g * L)
        s = jnp.zeros((L,), jnp.float32)
        for c in range(L):
            cv = jnp.full((L,), c, jnp.int32)
            s = s + plsc.load_gather(accs_v, [rows, cv])
        sum_v[pl.ds(g * L, L)] = s
    pltpu.sync_copy(sum_v, out_hbm.at[pl.ds(base, ROWS)])


def _tc_nll_body(scores_ref, ns_ref, out_ref):
    nll = jnp.mean(jnp.log(ns_ref[...])) - jnp.mean(scores_ref[...])
    out_ref[...] = jnp.broadcast_to(nll, (1, 1))


_tc_nll = pl.pallas_call(
    _tc_nll_body,
    out_shape=jax.ShapeDtypeStruct((1, 1), jnp.float32),
)


def kernel(center_words, target_words, all_vocabs, emb_v, emb_u):
    cen_t = jnp.swapaxes(center_words, 0, 1)
    tgt_t = jnp.swapaxes(target_words, 0, 1)
    emb_v_t = jnp.swapaxes(emb_v, 0, 1)
    emb_u_t = jnp.swapaxes(emb_u, 0, 1)
    av_t = jnp.swapaxes(all_vocabs, 0, 1)
    e_s, av_s, scores = _tc_scores(cen_t, tgt_t, emb_v_t, emb_u_t, av_t)
    norm_sum = _sc_gather_sum(e_s, av_s)
    return _tc_nll(scores, norm_sum)[0, 0]


# speedup vs baseline: 1.0508x; 1.0508x over previous
import functools

import jax
import jax.numpy as jnp
from jax import lax
from jax.experimental import pallas as pl
from jax.experimental.pallas import tpu as pltpu
from jax.experimental.pallas import tpu_sc as plsc

B = 1024
V = 1000
E = 64
NT = 8
SW = 128
NC = 2
NS = 16
L = 16
NW = NC * NS
ROWS = B // NW


def _tc_scores_body(cen_ref, tgt_ref, emb_v_ref, emb_u_ref, av_ref,
                    e_out_ref, av_out_ref, scores_ref):
    rowv = lax.broadcasted_iota(jnp.int32, (V, B), 0)
    oh_c = (cen_ref[...] == rowv).astype(jnp.float32)
    c_t = jnp.dot(emb_v_ref[...], oh_c,
                  preferred_element_type=jnp.float32)
    m_t = lax.dot_general(emb_u_ref[...], c_t, (((0,), (0,)), ((), ())),
                          preferred_element_type=jnp.float32)
    scores_ref[...] = jnp.sum(jnp.where(tgt_ref[...] == rowv, m_t, 0.0),
                              axis=0, keepdims=True)
    e_bt = jnp.transpose(jnp.exp(m_t), (1, 0))
    av_bt = jnp.transpose(av_ref[...], (1, 0))
    for t in range(NT):
        w = min(SW, V - t * SW)
        e_out_ref[t, :, :w] = e_bt[:, t * SW:t * SW + w]
        av_out_ref[t, :, :w] = av_bt[:, t * SW:t * SW + w]
    e_out_ref[NT - 1, :, V - (NT - 1) * SW:] = jnp.zeros(
        (B, NT * SW - V), jnp.float32)
    av_out_ref[NT - 1, :, V - (NT - 1) * SW:] = jnp.full(
        (B, NT * SW - V), V, jnp.int32)


_tc_scores = pl.pallas_call(
    _tc_scores_body,
    out_shape=(
        jax.ShapeDtypeStruct((NT, B, SW), jnp.float32),
        jax.ShapeDtypeStruct((NT, B, SW), jnp.int32),
        jax.ShapeDtypeStruct((1, B), jnp.float32),
    ),
)


_sc_mesh = plsc.VectorSubcoreMesh(core_axis_name="c", subcore_axis_name="s")


@functools.partial(
    pl.kernel,
    out_type=jax.ShapeDtypeStruct((B,), jnp.float32),
    mesh=_sc_mesh,
    compiler_params=pltpu.CompilerParams(
        use_tc_tiling_on_sc=False, needs_layout_passes=False),
    scratch_types=[
        pltpu.VMEM((NT, ROWS, SW), jnp.float32),
        pltpu.VMEM((NT, ROWS, SW), jnp.int32),
        pltpu.VMEM((ROWS, L), jnp.float32),
        pltpu.VMEM((ROWS,), jnp.float32),
        pltpu.SemaphoreType.DMA,
        pltpu.SemaphoreType.DMA,
    ],
)
def _sc_gather_sum(e_hbm, av_hbm, out_hbm, e_v, av_v, accs_v, sum_v,
                   sem_e, sem_i):
    wid = lax.axis_index("s") * NC + lax.axis_index("c")
    base = wid * ROWS
    cp_e = pltpu.async_copy(e_hbm.at[:, pl.ds(base, ROWS)], e_v, sem_e)
    cp_i = pltpu.async_copy(av_hbm.at[:, pl.ds(base, ROWS)], av_v, sem_i)
    cp_e.wait()
    cp_i.wait()

    def row_body(r, carry):
        rsplat = jnp.zeros((L,), jnp.int32) + r

        def chunk(k, a, _r=r, _rs=rsplat):
            cols = av_v[k // (SW // L), _r, pl.ds((k % (SW // L)) * L, L)]
            tt = cols >> 7
            cc = cols & 127
            return a + plsc.load_gather(e_v, [tt, _rs, cc])

        acc = lax.fori_loop(0, NT * (SW // L), chunk,
                            jnp.zeros((L,), jnp.float32), unroll=8)
        accs_v[r, :] = acc
        return carry

    lax.fori_loop(0, ROWS, row_body, 0)
    for g in range(ROWS // L):
        rows = lax.iota(jnp.int32, L) + (g * L)
        s = jnp.zeros((L,), jnp.float32)
        for c in range(L):
            cv = jnp.full((L,), c, jnp.int32)
            s = s + plsc.load_gather(accs_v, [rows, cv])
        sum_v[pl.ds(g * L, L)] = s
    pltpu.sync_copy(sum_v, out_hbm.at[pl.ds(base, ROWS)])


def _tc_nll_body(scores_ref, ns_ref, out_ref):
    nll = jnp.mean(jnp.log(ns_ref[...])) - jnp.mean(scores_ref[...])
    out_ref[...] = jnp.broadcast_to(nll, (1, 1))


_tc_nll = pl.pallas_call(
    _tc_nll_body,
    out_shape=jax.ShapeDtypeStruct((1, 1), jnp.float32),
)


def kernel(center_words, target_words, all_vocabs, emb_v, emb_u):
    cen_t = jnp.swapaxes(center_words, 0, 1)
    tgt_t = jnp.swapaxes(target_words, 0, 1)
    emb_v_t = jnp.swapaxes(emb_v, 0, 1)
    emb_u_t = jnp.swapaxes(emb_u, 0, 1)
    av_t = jnp.swapaxes(all_vocabs, 0, 1)
    e_s, av_s, scores = _tc_scores(cen_t, tgt_t, emb_v_t, emb_u_t, av_t)
    norm_sum = _sc_gather_sum(e_s, av_s)
    return _tc_nll(scores, norm_sum)[0, 0]


# speedup vs baseline: 1.0790x; 1.0268x over previous
import functools

import jax
import jax.numpy as jnp
from jax import lax
from jax.experimental import pallas as pl
from jax.experimental.pallas import tpu as pltpu
from jax.experimental.pallas import tpu_sc as plsc

B = 1024
V = 1000
E = 64
NT = 8
SW = 128
NC = 2
NS = 16
L = 16
NW = NC * NS
ROWS = B // NW


def _tc_scores_body(cen_ref, tgt_ref, emb_v_ref, emb_u_ref, av_ref,
                    e_out_ref, av_out_ref, scores_ref):
    rowv = lax.broadcasted_iota(jnp.int32, (V, B), 0)
    oh_c = (cen_ref[...] == rowv).astype(jnp.float32)
    c_t = jnp.dot(emb_v_ref[...], oh_c,
                  preferred_element_type=jnp.float32)
    m_t = lax.dot_general(emb_u_ref[...], c_t, (((0,), (0,)), ((), ())),
                          preferred_element_type=jnp.float32)
    scores_ref[...] = jnp.sum(jnp.where(tgt_ref[...] == rowv, m_t, 0.0),
                              axis=0, keepdims=True)
    e_bt = jnp.transpose(jnp.exp(m_t), (1, 0))
    av_bt = jnp.transpose(av_ref[...], (1, 0))
    for t in range(NT):
        w = min(SW, V - t * SW)
        e_out_ref[t, :, :w] = e_bt[:, t * SW:t * SW + w]
        av_out_ref[t, :, :w] = av_bt[:, t * SW:t * SW + w]
    e_out_ref[NT - 1, :, V - (NT - 1) * SW:] = jnp.zeros(
        (B, NT * SW - V), jnp.float32)
    av_out_ref[NT - 1, :, V - (NT - 1) * SW:] = jnp.full(
        (B, NT * SW - V), V, jnp.int32)


_tc_scores = pl.pallas_call(
    _tc_scores_body,
    out_shape=(
        jax.ShapeDtypeStruct((NT, B, SW), jnp.float32),
        jax.ShapeDtypeStruct((NT, B, SW), jnp.int32),
        jax.ShapeDtypeStruct((1, B), jnp.float32),
    ),
)


_sc_mesh = plsc.VectorSubcoreMesh(core_axis_name="c", subcore_axis_name="s")


@functools.partial(
    pl.kernel,
    out_type=jax.ShapeDtypeStruct((B,), jnp.float32),
    mesh=_sc_mesh,
    compiler_params=pltpu.CompilerParams(
        use_tc_tiling_on_sc=False, needs_layout_passes=False),
    scratch_types=[
        pltpu.VMEM((NT, ROWS, SW), jnp.float32),
        pltpu.VMEM((NT, ROWS, SW), jnp.int32),
        pltpu.VMEM((ROWS, L), jnp.float32),
        pltpu.VMEM((ROWS,), jnp.float32),
        pltpu.SemaphoreType.DMA,
        pltpu.SemaphoreType.DMA,
    ],
)
def _sc_gather_sum(e_hbm, av_hbm, out_hbm, e_v, av_v, accs_v, sum_v,
                   sem_e, sem_i):
    wid = lax.axis_index("s") * NC + lax.axis_index("c")
    base = wid * ROWS
    half = ROWS // 2
    cp1 = [pltpu.async_copy(e_hbm.at[:, pl.ds(base, half)],
                            e_v.at[:, pl.ds(0, half)], sem_e),
           pltpu.async_copy(av_hbm.at[:, pl.ds(base, half)],
                            av_v.at[:, pl.ds(0, half)], sem_i)]
    cp2 = [pltpu.async_copy(e_hbm.at[:, pl.ds(base + half, half)],
                            e_v.at[:, pl.ds(half, half)], sem_e),
           pltpu.async_copy(av_hbm.at[:, pl.ds(base + half, half)],
                            av_v.at[:, pl.ds(half, half)], sem_i)]

    def row_body(r, carry):
        rsplat = jnp.zeros((L,), jnp.int32) + r

        def chunk(k, a, _r=r, _rs=rsplat):
            cols = av_v[k // (SW // L), _r, pl.ds((k % (SW // L)) * L, L)]
            tt = cols >> 7
            cc = cols & 127
            return a + plsc.load_gather(e_v, [tt, _rs, cc])

        acc = lax.fori_loop(0, NT * (SW // L), chunk,
                            jnp.zeros((L,), jnp.float32), unroll=8)
        accs_v[r, :] = acc
        return carry

    for cp in cp1:
        cp.wait()
    lax.fori_loop(0, half, row_body, 0)
    for cp in cp2:
        cp.wait()
    lax.fori_loop(half, ROWS, row_body, 0)
    for g in range(ROWS // L):
        rows = lax.iota(jnp.int32, L) + (g * L)
        s = jnp.zeros((L,), jnp.float32)
        for c in range(L):
            cv = jnp.full((L,), c, jnp.int32)
            s = s + plsc.load_gather(accs_v, [rows, cv])
        sum_v[pl.ds(g * L, L)] = s
    pltpu.sync_copy(sum_v, out_hbm.at[pl.ds(base, ROWS)])


def _tc_nll_body(scores_ref, ns_ref, out_ref):
    nll = jnp.mean(jnp.log(ns_ref[...])) - jnp.mean(scores_ref[...])
    out_ref[...] = jnp.broadcast_to(nll, (1, 1))


_tc_nll = pl.pallas_call(
    _tc_nll_body,
    out_shape=jax.ShapeDtypeStruct((1, 1), jnp.float32),
)


def kernel(center_words, target_words, all_vocabs, emb_v, emb_u):
    cen_t = jnp.swapaxes(center_words, 0, 1)
    tgt_t = jnp.swapaxes(target_words, 0, 1)
    emb_v_t = jnp.swapaxes(emb_v, 0, 1)
    emb_u_t = jnp.swapaxes(emb_u, 0, 1)
    av_t = jnp.swapaxes(all_vocabs, 0, 1)
    e_s, av_s, scores = _tc_scores(cen_t, tgt_t, emb_v_t, emb_u_t, av_t)
    norm_sum = _sc_gather_sum(e_s, av_s)
    return _tc_nll(scores, norm_sum)[0, 0]
